# Initial kernel scaffold; baseline (speedup 1.0000x reference)
import jax
import jax.numpy as jnp
from jax.experimental import pallas as pl


def kernel(y, counts):
    raise NotImplementedError("write your pallas kernel here")


# speedup vs baseline: 65.3966x; 65.3966x over previous
import functools

import jax
import jax.numpy as jnp
from jax import lax
from jax.experimental import pallas as pl
from jax.experimental.pallas import tpu as pltpu
from jax.experimental.pallas import tpu_sc as plsc

N_TOTAL = 8388608
NUM_CLASSES = 2
BETA = 0.99

NC = 2
NS = 16
NW = NC * NS
PER_W = N_TOTAL // NW
CHUNK = 32768
NCH = PER_W // CHUNK
NBUF = 2
LANES = 16
UNROLL = 4


def _sc_partial_sums_body(y_hbm, part_hbm, buf0, buf1, acc_v, sem0, sem1):
    c = lax.axis_index("c")
    s = lax.axis_index("s")
    wid = s * NC + c
    base = wid * PER_W

    bufs = (buf0, buf1)
    sems = (sem0, sem1)

    for b in range(NBUF):
        pltpu.make_async_copy(
            y_hbm.at[pl.ds(base + b * CHUNK, CHUNK)], bufs[b], sems[b]
        ).start()

    accs = tuple(jnp.zeros((LANES,), jnp.int32) for _ in range(UNROLL))

    for i in range(NCH):
        slot = i % NBUF
        pltpu.make_async_copy(
            y_hbm.at[pl.ds(base + i * CHUNK, CHUNK)], bufs[slot], sems[slot]
        ).wait()
        bref = bufs[slot]

        def body(j, accs, bref=bref):
            off = j * (LANES * UNROLL)
            return tuple(
                a + bref[pl.ds(off + k * LANES, LANES)]
                for k, a in enumerate(accs)
            )

        accs = lax.fori_loop(0, CHUNK // (LANES * UNROLL), body, accs)

        nxt = i + NBUF
        if nxt < NCH:
            pltpu.make_async_copy(
                y_hbm.at[pl.ds(base + nxt * CHUNK, CHUNK)], bufs[slot], sems[slot]
            ).start()

    acc = accs[0]
    for a in accs[1:]:
        acc = acc + a
    acc_v[...] = acc
    pltpu.sync_copy(acc_v, part_hbm.at[pl.ds(wid * LANES, LANES)])


@functools.cache
def _sc_partial_sums():
    return pl.kernel(
        _sc_partial_sums_body,
        out_type=jax.ShapeDtypeStruct((NW * LANES,), jnp.int32),
        mesh=plsc.VectorSubcoreMesh(
            core_axis_name="c", subcore_axis_name="s", num_cores=NC, num_subcores=NS
        ),
        scratch_types=[
            pltpu.VMEM((CHUNK,), jnp.int32),
            pltpu.VMEM((CHUNK,), jnp.int32),
            pltpu.VMEM((LANES,), jnp.int32),
            pltpu.SemaphoreType.DMA,
            pltpu.SemaphoreType.DMA,
        ],
    )


def _tc_finalize_body(part_ref, counts_ref, nc_ref, w_ref):
    total_pos = jnp.sum(part_ref[...]).astype(jnp.float32)
    neg = jnp.float32(N_TOTAL) - total_pos
    c0 = counts_ref[0]
    c1 = counts_ref[1]
    n0 = BETA * c0 + (1.0 - BETA) * neg
    n1 = BETA * c1 + (1.0 - BETA) * total_pos
    s0 = n0 + 1.0
    s1 = n1 + 1.0
    tot = s0 + s1
    w0 = tot / (NUM_CLASSES * s0)
    w1 = tot / (NUM_CLASSES * s1)
    wm = (w0 + w1) * 0.5
    w0n = w0 / (wm + 1e-8)
    w1n = w1 / (wm + 1e-8)
    idx = lax.broadcasted_iota(jnp.int32, (1, 128), 1)
    is0 = idx == 0
    is1 = idx == 1
    nc_ref[...] = jnp.where(is0, n0, jnp.where(is1, n1, 0.0))
    w_ref[...] = jnp.where(is0, w0n, jnp.where(is1, w1n, 0.0))


_tc_finalize = pl.pallas_call(
    _tc_finalize_body,
    out_shape=(
        jax.ShapeDtypeStruct((1, 128), jnp.float32),
        jax.ShapeDtypeStruct((1, 128), jnp.float32),
    ),
    in_specs=[
        pl.BlockSpec(memory_space=pltpu.VMEM),
        pl.BlockSpec(memory_space=pltpu.SMEM),
    ],
)


@jax.jit
def kernel(y, counts):
    y = y.astype(jnp.int32)
    part = _sc_partial_sums()(y)
    nc_pad, w_pad = _tc_finalize(part.reshape(4, 128), counts)
    return nc_pad[0, :NUM_CLASSES], w_pad[0, :NUM_CLASSES]
